# Initial kernel scaffold; baseline (speedup 1.0000x reference)
import jax
import jax.numpy as jnp
from jax.experimental import pallas as pl


def kernel(x, edge_index, W1, b1, W2, b2):
    raise NotImplementedError("write your pallas kernel here")


# speedup vs baseline: 21.4428x; 21.4428x over previous
import functools

import jax
import jax.numpy as jnp
from jax import lax
from jax.experimental import pallas as pl
from jax.experimental.pallas import tpu as pltpu
from jax.experimental.pallas import tpu_sc as plsc

N_NODES = 10000
N_PAD = 10112
PAD_IDX = N_NODES
CHUNK = 128
N_CORES = 2
N_SUB = 16
N_WORKERS = N_CORES * N_SUB
ROWS_PER_SUB = N_PAD // N_SUB


def _sc_agg_body(ch_per_w, hs_hbm, src_hbm, dst_hbm, zero_hbm, out_hbm,
                 sidx, didx, rows0, rows1, acc, sem0, sem1):
    cid = lax.axis_index("c")
    sid = lax.axis_index("s")
    wid = cid * N_SUB + sid

    pltpu.sync_copy(zero_hbm.at[pl.ds(sid * ROWS_PER_SUB, ROWS_PER_SUB)],
                    acc.at[pl.ds(sid * ROWS_PER_SUB, ROWS_PER_SUB)])
    base = wid * ch_per_w
    pltpu.sync_copy(src_hbm.at[pl.ds(base, ch_per_w)], sidx)
    pltpu.sync_copy(dst_hbm.at[pl.ds(base, ch_per_w)], didx)
    plsc.subcore_barrier()

    @pl.loop(0, ch_per_w, step=2)
    def _(j):
        pltpu.async_copy(hs_hbm.at[sidx.at[j]], rows0, sem0)
        pltpu.async_copy(hs_hbm.at[sidx.at[j + 1]], rows1, sem1)
        pltpu.make_async_copy(hs_hbm.at[sidx.at[j]], rows0, sem0).wait()
        pltpu.sync_copy(rows0, acc.at[didx.at[j]], add=True)
        pltpu.make_async_copy(hs_hbm.at[sidx.at[j + 1]], rows1, sem1).wait()
        pltpu.sync_copy(rows1, acc.at[didx.at[j + 1]], add=True)

    plsc.subcore_barrier()
    pltpu.sync_copy(acc.at[pl.ds(sid * ROWS_PER_SUB, ROWS_PER_SUB)],
                    out_hbm.at[cid, pl.ds(sid * ROWS_PER_SUB, ROWS_PER_SUB)])


def _make_sc_agg(n_chunks, d):
    ch_per_w = n_chunks // N_WORKERS
    mesh = plsc.VectorSubcoreMesh(core_axis_name="c", subcore_axis_name="s")
    return pl.kernel(
        functools.partial(_sc_agg_body, ch_per_w),
        out_type=jax.ShapeDtypeStruct((N_CORES, N_PAD, d), jnp.float32),
        mesh=mesh,
        scratch_types=[
            pltpu.VMEM((ch_per_w, CHUNK), jnp.int32),
            pltpu.VMEM((ch_per_w, CHUNK), jnp.int32),
            pltpu.VMEM((CHUNK, d), jnp.float32),
            pltpu.VMEM((CHUNK, d), jnp.float32),
            pltpu.VMEM_SHARED((N_PAD, d), jnp.float32),
            pltpu.SemaphoreType.DMA,
            pltpu.SemaphoreType.DMA,
        ],
        compiler_params=pltpu.CompilerParams(use_tc_tiling_on_sc=False),
    )


def _tc_matmul_body(x_ref, w_ref, o_ref):
    o_ref[...] = jnp.dot(x_ref[...], w_ref[...],
                         preferred_element_type=jnp.float32)


def _tc_scale_body(deg_ref, h_ref, hs_ref, dinv_ref):
    deg = deg_ref[0, :, 0:1] + deg_ref[1, :, 0:1] + 1.0
    dinv = lax.rsqrt(deg)
    dinv_ref[...] = dinv
    hs_ref[...] = h_ref[...] * dinv


def _tc_mid_body(acc_ref, hs_ref, dinv_ref, b1_ref, w2_ref, o_ref):
    dinv = dinv_ref[...]
    out1 = (acc_ref[0] + acc_ref[1] + hs_ref[...]) * dinv + b1_ref[...]
    out1 = jnp.maximum(out1, 0.0)
    h2 = jnp.dot(out1, w2_ref[...], preferred_element_type=jnp.float32)
    row = lax.broadcasted_iota(jnp.int32, (N_PAD, 1), 0)
    o_ref[...] = jnp.where(row < N_NODES, h2 * dinv, 0.0)


def _tc_out_body(acc_ref, hs_ref, dinv_ref, b2_ref, o_ref):
    full = (acc_ref[0] + acc_ref[1] + hs_ref[...]) * dinv_ref[...]
    logits = full[:N_NODES, :7] + b2_ref[...]
    m = jnp.max(logits, axis=1, keepdims=True)
    e = jnp.exp(logits - m)
    o_ref[...] = e / jnp.sum(e, axis=1, keepdims=True)


def kernel(x, edge_index, W1, b1, W2, b2):
    x = x.astype(jnp.float32)
    src = edge_index[0].astype(jnp.int32)
    dst = edge_index[1].astype(jnp.int32)
    n_edges = src.shape[0]

    per_w = CHUNK * ((n_edges + N_WORKERS * CHUNK - 1) // (N_WORKERS * CHUNK))
    if (per_w // CHUNK) % 2:
        per_w += CHUNK
    e_pad = per_w * N_WORKERS
    n_chunks = e_pad // CHUNK
    pad = jnp.full((e_pad - n_edges,), PAD_IDX, jnp.int32)
    src_c = jnp.concatenate([src, pad]).reshape(n_chunks, CHUNK)
    dst_c = jnp.concatenate([dst, pad]).reshape(n_chunks, CHUNK)

    x_p = jnp.pad(x, ((0, N_PAD - N_NODES), (0, 0)))
    w2_p = jnp.pad(W2.astype(jnp.float32), ((0, 0), (0, 16 - W2.shape[1])))
    ones_t = jnp.pad(jnp.ones((N_NODES, 16), jnp.float32),
                     ((0, N_PAD - N_NODES), (0, 0)))
    zeros16 = jnp.zeros((N_PAD, 16), jnp.float32)
    zeros32 = jnp.zeros((N_PAD, 32), jnp.float32)

    agg16 = _make_sc_agg(n_chunks, 16)
    agg32 = _make_sc_agg(n_chunks, 32)

    deg_parts = agg16(ones_t, src_c, dst_c, zeros16)
    h1 = pl.pallas_call(
        _tc_matmul_body,
        out_shape=jax.ShapeDtypeStruct((N_PAD, 32), jnp.float32),
    )(x_p, W1.astype(jnp.float32))

    hs1, dinv = pl.pallas_call(
        _tc_scale_body,
        out_shape=(jax.ShapeDtypeStruct((N_PAD, 32), jnp.float32),
                   jax.ShapeDtypeStruct((N_PAD, 1), jnp.float32)),
    )(deg_parts, h1)

    acc1 = agg32(hs1, src_c, dst_c, zeros32)

    hs2 = pl.pallas_call(
        _tc_mid_body,
        out_shape=jax.ShapeDtypeStruct((N_PAD, 16), jnp.float32),
    )(acc1, hs1, dinv, b1.astype(jnp.float32).reshape(1, 32), w2_p)

    acc2 = agg16(hs2, src_c, dst_c, zeros16)

    out = pl.pallas_call(
        _tc_out_body,
        out_shape=jax.ShapeDtypeStruct((N_NODES, 7), jnp.float32),
    )(acc2, hs2, dinv, b2.astype(jnp.float32).reshape(1, 7))
    return out


# speedup vs baseline: 41.8500x; 1.9517x over previous
import functools

import jax
import jax.numpy as jnp
from jax import lax
from jax.experimental import pallas as pl
from jax.experimental.pallas import tpu as pltpu
from jax.experimental.pallas import tpu_sc as plsc

N_NODES = 10000
N_PAD = 10112
PAD_IDX = N_NODES
CHUNK = 128
N_CORES = 2
N_SUB = 16
N_WORKERS = N_CORES * N_SUB
ROWS_PER_SUB = N_PAD // N_SUB


def _sc_agg_body(ch_per_w, hs_hbm, src_hbm, dst_hbm, zero_hbm, out_hbm,
                 sidx, didx, rows0, rows1, acc, sem0, sem1):
    cid = lax.axis_index("c")
    sid = lax.axis_index("s")
    wid = cid * N_SUB + sid

    pltpu.sync_copy(zero_hbm.at[pl.ds(sid * ROWS_PER_SUB, ROWS_PER_SUB)],
                    acc.at[pl.ds(sid * ROWS_PER_SUB, ROWS_PER_SUB)])
    base = wid * ch_per_w
    pltpu.sync_copy(src_hbm.at[pl.ds(base, ch_per_w)], sidx)
    pltpu.sync_copy(dst_hbm.at[pl.ds(base, ch_per_w)], didx)
    plsc.subcore_barrier()

    @pl.loop(0, ch_per_w, step=2)
    def _(j):
        pltpu.async_copy(hs_hbm.at[sidx.at[j]], rows0, sem0)
        pltpu.async_copy(hs_hbm.at[sidx.at[j + 1]], rows1, sem1)
        pltpu.make_async_copy(hs_hbm.at[sidx.at[j]], rows0, sem0).wait()
        pltpu.sync_copy(rows0, acc.at[didx.at[j]], add=True)
        pltpu.make_async_copy(hs_hbm.at[sidx.at[j + 1]], rows1, sem1).wait()
        pltpu.sync_copy(rows1, acc.at[didx.at[j + 1]], add=True)

    plsc.subcore_barrier()
    pltpu.sync_copy(acc.at[pl.ds(sid * ROWS_PER_SUB, ROWS_PER_SUB)],
                    out_hbm.at[cid, pl.ds(sid * ROWS_PER_SUB, ROWS_PER_SUB)])


def _make_sc_agg(n_chunks, d):
    ch_per_w = n_chunks // N_WORKERS
    mesh = plsc.VectorSubcoreMesh(core_axis_name="c", subcore_axis_name="s")
    return pl.kernel(
        functools.partial(_sc_agg_body, ch_per_w),
        out_type=jax.ShapeDtypeStruct((N_CORES, N_PAD, d), jnp.float32),
        mesh=mesh,
        scratch_types=[
            pltpu.VMEM((ch_per_w, CHUNK), jnp.int32),
            pltpu.VMEM((ch_per_w, CHUNK), jnp.int32),
            pltpu.VMEM((CHUNK, d), jnp.float32),
            pltpu.VMEM((CHUNK, d), jnp.float32),
            pltpu.VMEM_SHARED((N_PAD, d), jnp.float32),
            pltpu.SemaphoreType.DMA,
            pltpu.SemaphoreType.DMA,
        ],
        compiler_params=pltpu.CompilerParams(use_tc_tiling_on_sc=False),
    )


def _sc_deg_body(ch_per_w, dst_hbm, zero_hbm, out_hbm, didx, ones_v, acc):
    cid = lax.axis_index("c")
    sid = lax.axis_index("s")
    wid = cid * N_SUB + sid

    pltpu.sync_copy(zero_hbm.at[pl.ds(sid * ROWS_PER_SUB, ROWS_PER_SUB)],
                    acc.at[pl.ds(sid * ROWS_PER_SUB, ROWS_PER_SUB)])
    pltpu.sync_copy(dst_hbm.at[pl.ds(wid * ch_per_w, ch_per_w)], didx)

    @pl.loop(0, CHUNK)
    def _(i):
        ones_v[i, :] = jnp.full((16,), 1.0, jnp.float32)

    plsc.subcore_barrier()

    @pl.loop(0, ch_per_w)
    def _(j):
        pltpu.sync_copy(ones_v, acc.at[didx.at[j]], add=True)

    plsc.subcore_barrier()
    pltpu.sync_copy(acc.at[pl.ds(sid * ROWS_PER_SUB, ROWS_PER_SUB)],
                    out_hbm.at[cid, pl.ds(sid * ROWS_PER_SUB, ROWS_PER_SUB)])


def _make_sc_deg(n_chunks):
    ch_per_w = n_chunks // N_WORKERS
    mesh = plsc.VectorSubcoreMesh(core_axis_name="c", subcore_axis_name="s")
    return pl.kernel(
        functools.partial(_sc_deg_body, ch_per_w),
        out_type=jax.ShapeDtypeStruct((N_CORES, N_PAD, 16), jnp.float32),
        mesh=mesh,
        scratch_types=[
            pltpu.VMEM((ch_per_w, CHUNK), jnp.int32),
            pltpu.VMEM((CHUNK, 16), jnp.float32),
            pltpu.VMEM_SHARED((N_PAD, 16), jnp.float32),
        ],
        compiler_params=pltpu.CompilerParams(use_tc_tiling_on_sc=False),
    )


def _tc_matmul_body(x_ref, w_ref, o_ref):
    o_ref[...] = jnp.dot(x_ref[...], w_ref[...],
                         preferred_element_type=jnp.float32)


def _tc_scale_body(deg_ref, h_ref, hs_ref, dinv_ref):
    deg = deg_ref[0, :, 0:1] + deg_ref[1, :, 0:1] + 1.0
    dinv = lax.rsqrt(deg)
    dinv_ref[...] = dinv
    hs_ref[...] = h_ref[...] * dinv


def _tc_mid_body(acc_ref, hs_ref, dinv_ref, b1_ref, w2_ref, o_ref):
    dinv = dinv_ref[...]
    out1 = (acc_ref[0] + acc_ref[1] + hs_ref[...]) * dinv + b1_ref[...]
    out1 = jnp.maximum(out1, 0.0)
    h2 = jnp.dot(out1, w2_ref[...], preferred_element_type=jnp.float32)
    row = lax.broadcasted_iota(jnp.int32, (N_PAD, 1), 0)
    o_ref[...] = jnp.where(row < N_NODES, h2 * dinv, 0.0)


def _tc_out_body(acc_ref, hs_ref, dinv_ref, b2_ref, o_ref):
    full = (acc_ref[0] + acc_ref[1] + hs_ref[...]) * dinv_ref[...]
    logits = full[:N_NODES, :7] + b2_ref[...]
    m = jnp.max(logits, axis=1, keepdims=True)
    e = jnp.exp(logits - m)
    o_ref[...] = e / jnp.sum(e, axis=1, keepdims=True)


def kernel(x, edge_index, W1, b1, W2, b2):
    x = x.astype(jnp.float32)
    src = edge_index[0].astype(jnp.int32)
    dst = edge_index[1].astype(jnp.int32)
    n_edges = src.shape[0]

    per_w = CHUNK * ((n_edges + N_WORKERS * CHUNK - 1) // (N_WORKERS * CHUNK))
    if (per_w // CHUNK) % 2:
        per_w += CHUNK
    e_pad = per_w * N_WORKERS
    n_chunks = e_pad // CHUNK
    pad = PAD_IDX + (jnp.arange(e_pad - n_edges, dtype=jnp.int32)
                     % (N_PAD - N_NODES))
    src_c = jnp.concatenate([src, pad]).reshape(n_chunks, CHUNK)
    dst_c = jnp.concatenate([dst, pad]).reshape(n_chunks, CHUNK)

    x_p = jnp.pad(x, ((0, N_PAD - N_NODES), (0, 0)))
    w2_p = jnp.pad(W2.astype(jnp.float32), ((0, 0), (0, 16 - W2.shape[1])))
    zeros16 = jnp.zeros((N_PAD, 16), jnp.float32)
    zeros32 = jnp.zeros((N_PAD, 32), jnp.float32)

    agg16 = _make_sc_agg(n_chunks, 16)
    agg32 = _make_sc_agg(n_chunks, 32)

    deg_parts = _make_sc_deg(n_chunks)(dst_c, zeros16)
    h1 = pl.pallas_call(
        _tc_matmul_body,
        out_shape=jax.ShapeDtypeStruct((N_PAD, 32), jnp.float32),
    )(x_p, W1.astype(jnp.float32))

    hs1, dinv = pl.pallas_call(
        _tc_scale_body,
        out_shape=(jax.ShapeDtypeStruct((N_PAD, 32), jnp.float32),
                   jax.ShapeDtypeStruct((N_PAD, 1), jnp.float32)),
    )(deg_parts, h1)

    acc1 = agg32(hs1, src_c, dst_c, zeros32)

    hs2 = pl.pallas_call(
        _tc_mid_body,
        out_shape=jax.ShapeDtypeStruct((N_PAD, 16), jnp.float32),
    )(acc1, hs1, dinv, b1.astype(jnp.float32).reshape(1, 32), w2_p)

    acc2 = agg16(hs2, src_c, dst_c, zeros16)

    out = pl.pallas_call(
        _tc_out_body,
        out_shape=jax.ShapeDtypeStruct((N_NODES, 7), jnp.float32),
    )(acc2, hs2, dinv, b2.astype(jnp.float32).reshape(1, 7))
    return out


# speedup vs baseline: 51.5007x; 1.2306x over previous
import functools

import jax
import jax.numpy as jnp
from jax import lax
from jax.experimental import pallas as pl
from jax.experimental.pallas import tpu as pltpu
from jax.experimental.pallas import tpu_sc as plsc

N_NODES = 10000
N_PAD = 10112
PAD_IDX = N_NODES
CHUNK = 128
N_CORES = 2
N_SUB = 16
N_WORKERS = N_CORES * N_SUB
ROWS_PER_SUB = N_PAD // N_SUB


_NBUF = 4


def _sc_agg_body(ch_per_w, hs_hbm, src_hbm, dst_hbm, zero_hbm, out_hbm,
                 sidx, didx, rows, gsems, ssems, acc):
    cid = lax.axis_index("c")
    sid = lax.axis_index("s")
    wid = cid * N_SUB + sid

    pltpu.sync_copy(zero_hbm.at[pl.ds(sid * ROWS_PER_SUB, ROWS_PER_SUB)],
                    acc.at[pl.ds(sid * ROWS_PER_SUB, ROWS_PER_SUB)])
    base = wid * ch_per_w
    pltpu.sync_copy(src_hbm.at[pl.ds(base, ch_per_w)], sidx)
    pltpu.sync_copy(dst_hbm.at[pl.ds(base, ch_per_w)], didx)
    plsc.subcore_barrier()

    def gather(j, b):
        pltpu.async_copy(hs_hbm.at[sidx.at[j]], rows[b], gsems[b])

    def wait_gather(b):
        pltpu.make_async_copy(hs_hbm.at[sidx.at[0]], rows[b], gsems[b]).wait()

    def scatter(j, b):
        pltpu.async_copy(rows[b], acc.at[didx.at[j]], ssems[b], add=True)

    def wait_scatter(b):
        pltpu.make_async_copy(rows[b], acc.at[didx.at[0]], ssems[b]).wait()

    for b in range(_NBUF):
        gather(b, b)

    @pl.loop(0, ch_per_w - _NBUF, step=_NBUF)
    def _(j):
        for b in range(_NBUF):
            wait_gather(b)
            scatter(j + b, b)
        for b in range(_NBUF):
            wait_scatter(b)
            gather(j + _NBUF + b, b)

    for b in range(_NBUF):
        wait_gather(b)
        scatter(ch_per_w - _NBUF + b, b)
    for b in range(_NBUF):
        wait_scatter(b)

    plsc.subcore_barrier()
    pltpu.sync_copy(acc.at[pl.ds(sid * ROWS_PER_SUB, ROWS_PER_SUB)],
                    out_hbm.at[cid, pl.ds(sid * ROWS_PER_SUB, ROWS_PER_SUB)])


def _make_sc_agg(n_chunks, d):
    ch_per_w = n_chunks // N_WORKERS
    mesh = plsc.VectorSubcoreMesh(core_axis_name="c", subcore_axis_name="s")
    return pl.kernel(
        functools.partial(_sc_agg_body, ch_per_w),
        out_type=jax.ShapeDtypeStruct((N_CORES, N_PAD, d), jnp.float32),
        mesh=mesh,
        scratch_types=[
            pltpu.VMEM((ch_per_w, CHUNK), jnp.int32),
            pltpu.VMEM((ch_per_w, CHUNK), jnp.int32),
            [pltpu.VMEM((CHUNK, d), jnp.float32) for _ in range(_NBUF)],
            [pltpu.SemaphoreType.DMA for _ in range(_NBUF)],
            [pltpu.SemaphoreType.DMA for _ in range(_NBUF)],
            pltpu.VMEM_SHARED((N_PAD, d), jnp.float32),
        ],
        compiler_params=pltpu.CompilerParams(use_tc_tiling_on_sc=False),
    )


def _sc_deg_body(ch_per_w, dst_hbm, zero_hbm, out_hbm, didx, ones_v, acc):
    cid = lax.axis_index("c")
    sid = lax.axis_index("s")
    wid = cid * N_SUB + sid

    pltpu.sync_copy(zero_hbm.at[pl.ds(sid * ROWS_PER_SUB, ROWS_PER_SUB)],
                    acc.at[pl.ds(sid * ROWS_PER_SUB, ROWS_PER_SUB)])
    pltpu.sync_copy(dst_hbm.at[pl.ds(wid * ch_per_w, ch_per_w)], didx)

    @pl.loop(0, CHUNK)
    def _(i):
        ones_v[i, :] = jnp.full((16,), 1.0, jnp.float32)

    plsc.subcore_barrier()

    @pl.loop(0, ch_per_w)
    def _(j):
        pltpu.sync_copy(ones_v, acc.at[didx.at[j]], add=True)

    plsc.subcore_barrier()
    pltpu.sync_copy(acc.at[pl.ds(sid * ROWS_PER_SUB, ROWS_PER_SUB)],
                    out_hbm.at[cid, pl.ds(sid * ROWS_PER_SUB, ROWS_PER_SUB)])


def _make_sc_deg(n_chunks):
    ch_per_w = n_chunks // N_WORKERS
    mesh = plsc.VectorSubcoreMesh(core_axis_name="c", subcore_axis_name="s")
    return pl.kernel(
        functools.partial(_sc_deg_body, ch_per_w),
        out_type=jax.ShapeDtypeStruct((N_CORES, N_PAD, 16), jnp.float32),
        mesh=mesh,
        scratch_types=[
            pltpu.VMEM((ch_per_w, CHUNK), jnp.int32),
            pltpu.VMEM((CHUNK, 16), jnp.float32),
            pltpu.VMEM_SHARED((N_PAD, 16), jnp.float32),
        ],
        compiler_params=pltpu.CompilerParams(use_tc_tiling_on_sc=False),
    )


def _tc_matmul_body(x_ref, w_ref, o_ref):
    h = jnp.dot(x_ref[...], w_ref[...], preferred_element_type=jnp.float32)
    o_ref[...] = jnp.concatenate(
        [h, jnp.zeros((N_PAD - N_NODES, h.shape[1]), jnp.float32)], axis=0)


def _tc_scale_body(deg_ref, h_ref, hs_ref, dinv_ref):
    deg = deg_ref[0, :, 0:1] + deg_ref[1, :, 0:1] + 1.0
    dinv = lax.rsqrt(deg)
    dinv_ref[...] = dinv
    hs_ref[...] = h_ref[...] * dinv


def _tc_mid_body(acc_ref, hs_ref, dinv_ref, b1_ref, w2_ref, o_ref):
    dinv = dinv_ref[...]
    out1 = (acc_ref[0] + acc_ref[1] + hs_ref[...]) * dinv + b1_ref[...]
    out1 = jnp.maximum(out1, 0.0)
    h2 = jnp.dot(out1, w2_ref[...], preferred_element_type=jnp.float32)
    row = lax.broadcasted_iota(jnp.int32, (N_PAD, 1), 0)
    o_ref[...] = jnp.where(row < N_NODES, h2 * dinv, 0.0)


def _tc_out_body(acc_ref, hs_ref, dinv_ref, b2_ref, o_ref):
    full = (acc_ref[0] + acc_ref[1] + hs_ref[...]) * dinv_ref[...]
    logits = full[:N_NODES, :7] + b2_ref[...]
    m = jnp.max(logits, axis=1, keepdims=True)
    e = jnp.exp(logits - m)
    o_ref[...] = e / jnp.sum(e, axis=1, keepdims=True)


def kernel(x, edge_index, W1, b1, W2, b2):
    x = x.astype(jnp.float32)
    src = edge_index[0].astype(jnp.int32)
    dst = edge_index[1].astype(jnp.int32)
    n_edges = src.shape[0]

    per_w = CHUNK * ((n_edges + N_WORKERS * CHUNK - 1) // (N_WORKERS * CHUNK))
    if (per_w // CHUNK) % _NBUF:
        per_w += CHUNK * (_NBUF - (per_w // CHUNK) % _NBUF)
    e_pad = per_w * N_WORKERS
    n_chunks = e_pad // CHUNK
    pad = PAD_IDX + (jnp.arange(e_pad - n_edges, dtype=jnp.int32)
                     % (N_PAD - N_NODES))
    src_c = jnp.concatenate([src, pad]).reshape(n_chunks, CHUNK)
    dst_c = jnp.concatenate([dst, pad]).reshape(n_chunks, CHUNK)

    w2_p = jnp.pad(W2.astype(jnp.float32), ((0, 0), (0, 16 - W2.shape[1])))
    zeros16 = jnp.zeros((N_PAD, 16), jnp.float32)
    zeros32 = jnp.zeros((N_PAD, 32), jnp.float32)

    agg16 = _make_sc_agg(n_chunks, 16)
    agg32 = _make_sc_agg(n_chunks, 32)

    deg_parts = _make_sc_deg(n_chunks)(dst_c, zeros16)
    h1 = pl.pallas_call(
        _tc_matmul_body,
        out_shape=jax.ShapeDtypeStruct((N_PAD, 32), jnp.float32),
    )(x, W1.astype(jnp.float32))

    hs1, dinv = pl.pallas_call(
        _tc_scale_body,
        out_shape=(jax.ShapeDtypeStruct((N_PAD, 32), jnp.float32),
                   jax.ShapeDtypeStruct((N_PAD, 1), jnp.float32)),
    )(deg_parts, h1)

    acc1 = agg32(hs1, src_c, dst_c, zeros32)

    hs2 = pl.pallas_call(
        _tc_mid_body,
        out_shape=jax.ShapeDtypeStruct((N_PAD, 16), jnp.float32),
    )(acc1, hs1, dinv, b1.astype(jnp.float32).reshape(1, 32), w2_p)

    acc2 = agg16(hs2, src_c, dst_c, zeros16)

    out = pl.pallas_call(
        _tc_out_body,
        out_shape=jax.ShapeDtypeStruct((N_NODES, 7), jnp.float32),
    )(acc2, hs2, dinv, b2.astype(jnp.float32).reshape(1, 7))
    return out


# speedup vs baseline: 58.4325x; 1.1346x over previous
import functools

import jax
import jax.numpy as jnp
from jax import lax
from jax.experimental import pallas as pl
from jax.experimental.pallas import tpu as pltpu
from jax.experimental.pallas import tpu_sc as plsc

N_NODES = 10000
N_PAD = 10112
PAD_IDX = N_NODES
CHUNK = 128
N_CORES = 2
N_SUB = 16
N_WORKERS = N_CORES * N_SUB
ROWS_PER_SUB = N_PAD // N_SUB


_NBUF = 8


def _sc_agg_body(ch_per_w, hs_hbm, src_hbm, dst_hbm, zero_hbm, out_hbm,
                 sidx, didx, rows, gsems, ssems, acc):
    cid = lax.axis_index("c")
    sid = lax.axis_index("s")
    wid = cid * N_SUB + sid

    pltpu.sync_copy(zero_hbm.at[pl.ds(sid * ROWS_PER_SUB, ROWS_PER_SUB)],
                    acc.at[pl.ds(sid * ROWS_PER_SUB, ROWS_PER_SUB)])
    base = wid * ch_per_w
    pltpu.sync_copy(src_hbm.at[pl.ds(base, ch_per_w)], sidx)
    pltpu.sync_copy(dst_hbm.at[pl.ds(base, ch_per_w)], didx)
    plsc.subcore_barrier()

    def gather(j, b):
        pltpu.async_copy(hs_hbm.at[sidx.at[j]], rows[b], gsems[b])

    def wait_gather(b):
        pltpu.make_async_copy(hs_hbm.at[sidx.at[0]], rows[b], gsems[b]).wait()

    def scatter(j, b):
        pltpu.async_copy(rows[b], acc.at[didx.at[j]], ssems[b], add=True)

    def wait_scatter(b):
        pltpu.make_async_copy(rows[b], acc.at[didx.at[0]], ssems[b]).wait()

    for b in range(_NBUF):
        gather(b, b)

    @pl.loop(0, ch_per_w - _NBUF, step=_NBUF)
    def _(j):
        for b in range(_NBUF):
            wait_gather(b)
            scatter(j + b, b)
        for b in range(_NBUF):
            wait_scatter(b)
            gather(j + _NBUF + b, b)

    for b in range(_NBUF):
        wait_gather(b)
        scatter(ch_per_w - _NBUF + b, b)
    for b in range(_NBUF):
        wait_scatter(b)

    plsc.subcore_barrier()
    pltpu.sync_copy(acc.at[pl.ds(sid * ROWS_PER_SUB, ROWS_PER_SUB)],
                    out_hbm.at[cid, pl.ds(sid * ROWS_PER_SUB, ROWS_PER_SUB)])


def _make_sc_agg(n_chunks, d):
    ch_per_w = n_chunks // N_WORKERS
    mesh = plsc.VectorSubcoreMesh(core_axis_name="c", subcore_axis_name="s")
    return pl.kernel(
        functools.partial(_sc_agg_body, ch_per_w),
        out_type=jax.ShapeDtypeStruct((N_CORES, N_PAD, d), jnp.float32),
        mesh=mesh,
        scratch_types=[
            pltpu.VMEM((ch_per_w, CHUNK), jnp.int32),
            pltpu.VMEM((ch_per_w, CHUNK), jnp.int32),
            [pltpu.VMEM((CHUNK, d), jnp.float32) for _ in range(_NBUF)],
            [pltpu.SemaphoreType.DMA for _ in range(_NBUF)],
            [pltpu.SemaphoreType.DMA for _ in range(_NBUF)],
            pltpu.VMEM_SHARED((N_PAD, d), jnp.float32),
        ],
        compiler_params=pltpu.CompilerParams(use_tc_tiling_on_sc=False),
    )


def _sc_deg_body(ch_per_w, dst_hbm, zero_hbm, out_hbm, didx, ones_v, cbuf,
                 cvec, acc, sem):
    cid = lax.axis_index("c")
    sid = lax.axis_index("s")
    wid = cid * N_SUB + sid

    pltpu.sync_copy(zero_hbm.at[pl.ds(sid * ROWS_PER_SUB, ROWS_PER_SUB)],
                    acc.at[pl.ds(sid * ROWS_PER_SUB, ROWS_PER_SUB)])
    pltpu.sync_copy(dst_hbm.at[pl.ds(wid * ch_per_w, ch_per_w)], didx)

    @pl.loop(0, CHUNK)
    def _(i):
        ones_v[i, :] = jnp.full((16,), 1.0, jnp.float32)

    plsc.subcore_barrier()

    @pl.loop(0, ch_per_w)
    def _(j):
        pltpu.async_copy(ones_v, acc.at[didx.at[j]], sem, add=True)

    @pl.loop(0, ch_per_w)
    def _(j):
        pltpu.make_async_copy(ones_v, acc.at[didx.at[0]], sem).wait()

    plsc.subcore_barrier()
    pltpu.sync_copy(acc.at[pl.ds(sid * ROWS_PER_SUB, ROWS_PER_SUB)], cbuf)
    zero16 = jnp.zeros((16,), jnp.int32)
    base = lax.iota(jnp.int32, 16)

    @pl.loop(0, ROWS_PER_SUB, step=16)
    def _(i):
        v = plsc.load_gather(cbuf, [base + i, zero16])
        cvec.at[pl.ds(i, 16)][...] = v

    pltpu.sync_copy(cvec, out_hbm.at[cid, pl.ds(sid * ROWS_PER_SUB,
                                                ROWS_PER_SUB)])


def _make_sc_deg(n_chunks):
    ch_per_w = n_chunks // N_WORKERS
    mesh = plsc.VectorSubcoreMesh(core_axis_name="c", subcore_axis_name="s")
    return pl.kernel(
        functools.partial(_sc_deg_body, ch_per_w),
        out_type=jax.ShapeDtypeStruct((N_CORES, N_PAD), jnp.float32),
        mesh=mesh,
        scratch_types=[
            pltpu.VMEM((ch_per_w, CHUNK), jnp.int32),
            pltpu.VMEM((CHUNK, 16), jnp.float32),
            pltpu.VMEM((ROWS_PER_SUB, 16), jnp.float32),
            pltpu.VMEM((ROWS_PER_SUB,), jnp.float32),
            pltpu.VMEM_SHARED((N_PAD, 16), jnp.float32),
            pltpu.SemaphoreType.DMA,
        ],
        compiler_params=pltpu.CompilerParams(use_tc_tiling_on_sc=False,
                                             needs_layout_passes=False),
    )


def _tc_matmul_body(x_ref, w_ref, o_ref):
    h = jnp.dot(x_ref[...], w_ref[...], preferred_element_type=jnp.float32)
    o_ref[...] = jnp.concatenate(
        [h, jnp.zeros((N_PAD - N_NODES, h.shape[1]), jnp.float32)], axis=0)


def _tc_scale_body(deg_ref, h_ref, hs_ref, dinv_ref):
    d = deg_ref[...]
    deg_row = d[0:1, :] + d[1:2, :] + 1.0
    dinv = lax.rsqrt(deg_row).reshape(N_PAD, 1)
    dinv_ref[...] = dinv
    hs_ref[...] = h_ref[...] * dinv


def _tc_mid_body(acc_ref, hs_ref, dinv_ref, b1_ref, w2_ref, o_ref):
    dinv = dinv_ref[...]
    out1 = (acc_ref[0] + acc_ref[1] + hs_ref[...]) * dinv + b1_ref[...]
    out1 = jnp.maximum(out1, 0.0)
    h2 = jnp.dot(out1, w2_ref[...], preferred_element_type=jnp.float32)
    row = lax.broadcasted_iota(jnp.int32, (N_PAD, 1), 0)
    o_ref[...] = jnp.where(row < N_NODES, h2 * dinv, 0.0)


def _tc_out_body(acc_ref, hs_ref, dinv_ref, b2_ref, o_ref):
    full = (acc_ref[0] + acc_ref[1] + hs_ref[...]) * dinv_ref[...]
    logits = full[:N_NODES, :7] + b2_ref[...]
    m = jnp.max(logits, axis=1, keepdims=True)
    e = jnp.exp(logits - m)
    o_ref[...] = e / jnp.sum(e, axis=1, keepdims=True)


def kernel(x, edge_index, W1, b1, W2, b2):
    x = x.astype(jnp.float32)
    src = edge_index[0].astype(jnp.int32)
    dst = edge_index[1].astype(jnp.int32)
    n_edges = src.shape[0]

    per_w = CHUNK * ((n_edges + N_WORKERS * CHUNK - 1) // (N_WORKERS * CHUNK))
    if (per_w // CHUNK) % _NBUF:
        per_w += CHUNK * (_NBUF - (per_w // CHUNK) % _NBUF)
    e_pad = per_w * N_WORKERS
    n_chunks = e_pad // CHUNK
    pad = PAD_IDX + (jnp.arange(e_pad - n_edges, dtype=jnp.int32)
                     % (N_PAD - N_NODES))
    src_c = jnp.concatenate([src, pad]).reshape(n_chunks, CHUNK)
    dst_c = jnp.concatenate([dst, pad]).reshape(n_chunks, CHUNK)

    w2_p = jnp.pad(W2.astype(jnp.float32), ((0, 0), (0, 16 - W2.shape[1])))
    zeros16 = jnp.zeros((N_PAD, 16), jnp.float32)
    zeros32 = jnp.zeros((N_PAD, 32), jnp.float32)

    agg16 = _make_sc_agg(n_chunks, 16)
    agg32 = _make_sc_agg(n_chunks, 32)

    deg_parts = _make_sc_deg(n_chunks)(dst_c, zeros16)
    h1 = pl.pallas_call(
        _tc_matmul_body,
        out_shape=jax.ShapeDtypeStruct((N_PAD, 32), jnp.float32),
    )(x, W1.astype(jnp.float32))

    hs1, dinv = pl.pallas_call(
        _tc_scale_body,
        out_shape=(jax.ShapeDtypeStruct((N_PAD, 32), jnp.float32),
                   jax.ShapeDtypeStruct((N_PAD, 1), jnp.float32)),
    )(deg_parts, h1)

    acc1 = agg32(hs1, src_c, dst_c, zeros32)

    hs2 = pl.pallas_call(
        _tc_mid_body,
        out_shape=jax.ShapeDtypeStruct((N_PAD, 16), jnp.float32),
    )(acc1, hs1, dinv, b1.astype(jnp.float32).reshape(1, 32), w2_p)

    acc2 = agg16(hs2, src_c, dst_c, zeros16)

    out = pl.pallas_call(
        _tc_out_body,
        out_shape=jax.ShapeDtypeStruct((N_NODES, 7), jnp.float32),
    )(acc2, hs2, dinv, b2.astype(jnp.float32).reshape(1, 7))
    return out


# speedup vs baseline: 59.6544x; 1.0209x over previous
import functools

import jax
import jax.numpy as jnp
from jax import lax
from jax.experimental import pallas as pl
from jax.experimental.pallas import tpu as pltpu
from jax.experimental.pallas import tpu_sc as plsc

N_NODES = 10000
N_PAD = 10112
PAD_IDX = N_NODES
CHUNK = 128
N_CORES = 2
N_SUB = 16
N_WORKERS = N_CORES * N_SUB
ROWS_PER_SUB = N_PAD // N_SUB


_NBUF = 8


def _zero_acc_slice(zbuf, acc, sid, d):
    @pl.loop(0, CHUNK)
    def _(i):
        if d == 16:
            zbuf[i, :] = jnp.zeros((16,), jnp.float32)
        else:
            for c in range(d // 16):
                zbuf[i, pl.ds(c * 16, 16)] = jnp.zeros((16,), jnp.float32)

    r0 = sid * ROWS_PER_SUB
    nfull = ROWS_PER_SUB // CHUNK
    for k in range(nfull):
        pltpu.sync_copy(zbuf, acc.at[pl.ds(r0 + k * CHUNK, CHUNK)])
    rem = ROWS_PER_SUB - nfull * CHUNK
    if rem:
        pltpu.sync_copy(zbuf.at[pl.ds(0, rem)],
                        acc.at[pl.ds(r0 + nfull * CHUNK, rem)])


def _sc_agg_body(ch_per_w, d, hs_hbm, src_hbm, dst_hbm, out_hbm,
                 sidx, didx, rows, gsems, ssems, zbuf, acc):
    cid = lax.axis_index("c")
    sid = lax.axis_index("s")
    wid = cid * N_SUB + sid

    _zero_acc_slice(zbuf, acc, sid, d)
    base = wid * ch_per_w
    pltpu.sync_copy(src_hbm.at[pl.ds(base, ch_per_w)], sidx)
    pltpu.sync_copy(dst_hbm.at[pl.ds(base, ch_per_w)], didx)
    plsc.subcore_barrier()

    def gather(j, b):
        pltpu.async_copy(hs_hbm.at[sidx.at[j]], rows[b], gsems[b])

    def wait_gather(b):
        pltpu.make_async_copy(hs_hbm.at[sidx.at[0]], rows[b], gsems[b]).wait()

    def scatter(j, b):
        pltpu.async_copy(rows[b], acc.at[didx.at[j]], ssems[b], add=True)

    def wait_scatter(b):
        pltpu.make_async_copy(rows[b], acc.at[didx.at[0]], ssems[b]).wait()

    for b in range(_NBUF):
        gather(b, b)

    @pl.loop(0, ch_per_w - _NBUF, step=_NBUF)
    def _(j):
        for b in range(_NBUF):
            wait_gather(b)
            scatter(j + b, b)
        for b in range(_NBUF):
            wait_scatter(b)
            gather(j + _NBUF + b, b)

    for b in range(_NBUF):
        wait_gather(b)
        scatter(ch_per_w - _NBUF + b, b)
    for b in range(_NBUF):
        wait_scatter(b)

    plsc.subcore_barrier()
    pltpu.sync_copy(acc.at[pl.ds(sid * ROWS_PER_SUB, ROWS_PER_SUB)],
                    out_hbm.at[pl.ds(sid * ROWS_PER_SUB, ROWS_PER_SUB),
                               pl.ds(cid * d, d)])


def _make_sc_agg(n_chunks, d):
    ch_per_w = n_chunks // N_WORKERS
    mesh = plsc.VectorSubcoreMesh(core_axis_name="c", subcore_axis_name="s")
    return pl.kernel(
        functools.partial(_sc_agg_body, ch_per_w, d),
        name=f"sc_agg{d}",
        out_type=jax.ShapeDtypeStruct((N_PAD, 2 * d), jnp.float32),
        mesh=mesh,
        scratch_types=[
            pltpu.VMEM((ch_per_w, CHUNK), jnp.int32),
            pltpu.VMEM((ch_per_w, CHUNK), jnp.int32),
            [pltpu.VMEM((CHUNK, d), jnp.float32) for _ in range(_NBUF)],
            [pltpu.SemaphoreType.DMA for _ in range(_NBUF)],
            [pltpu.SemaphoreType.DMA for _ in range(_NBUF)],
            pltpu.VMEM((CHUNK, d), jnp.float32),
            pltpu.VMEM_SHARED((N_PAD, d), jnp.float32),
        ],
        compiler_params=pltpu.CompilerParams(use_tc_tiling_on_sc=False,
                                             needs_layout_passes=False),
    )


def _sc_deg_body(ch_per_w, dst_hbm, out_hbm, didx, ones_v, zbuf, cbuf,
                 cvec, acc, sem):
    cid = lax.axis_index("c")
    sid = lax.axis_index("s")
    wid = cid * N_SUB + sid

    _zero_acc_slice(zbuf, acc, sid, 16)
    pltpu.sync_copy(dst_hbm.at[pl.ds(wid * ch_per_w, ch_per_w)], didx)

    @pl.loop(0, CHUNK)
    def _(i):
        ones_v[i, :] = jnp.full((16,), 1.0, jnp.float32)

    plsc.subcore_barrier()

    @pl.loop(0, ch_per_w)
    def _(j):
        pltpu.async_copy(ones_v, acc.at[didx.at[j]], sem, add=True)

    @pl.loop(0, ch_per_w)
    def _(j):
        pltpu.make_async_copy(ones_v, acc.at[didx.at[0]], sem).wait()

    plsc.subcore_barrier()
    pltpu.sync_copy(acc.at[pl.ds(sid * ROWS_PER_SUB, ROWS_PER_SUB)], cbuf)
    zero16 = jnp.zeros((16,), jnp.int32)
    base = lax.iota(jnp.int32, 16)

    @pl.loop(0, ROWS_PER_SUB, step=16)
    def _(i):
        v = plsc.load_gather(cbuf, [base + i, zero16])
        cvec.at[pl.ds(i, 16)][...] = v

    pltpu.sync_copy(cvec, out_hbm.at[cid, pl.ds(sid * ROWS_PER_SUB,
                                                ROWS_PER_SUB)])


def _make_sc_deg(n_chunks):
    ch_per_w = n_chunks // N_WORKERS
    mesh = plsc.VectorSubcoreMesh(core_axis_name="c", subcore_axis_name="s")
    return pl.kernel(
        functools.partial(_sc_deg_body, ch_per_w),
        out_type=jax.ShapeDtypeStruct((N_CORES, N_PAD), jnp.float32),
        mesh=mesh,
        scratch_types=[
            pltpu.VMEM((ch_per_w, CHUNK), jnp.int32),
            pltpu.VMEM((CHUNK, 16), jnp.float32),
            pltpu.VMEM((CHUNK, 16), jnp.float32),
            pltpu.VMEM((ROWS_PER_SUB, 16), jnp.float32),
            pltpu.VMEM((ROWS_PER_SUB,), jnp.float32),
            pltpu.VMEM_SHARED((N_PAD, 16), jnp.float32),
            pltpu.SemaphoreType.DMA,
        ],
        compiler_params=pltpu.CompilerParams(use_tc_tiling_on_sc=False,
                                             needs_layout_passes=False),
    )


def _tc_matmul_body(x_ref, w_ref, o_ref):
    h = jnp.dot(x_ref[...], w_ref[...], preferred_element_type=jnp.float32)
    o_ref[...] = jnp.concatenate(
        [h, jnp.zeros((N_PAD - N_NODES, h.shape[1]), jnp.float32)], axis=0)


def _tc_scale_body(deg_ref, h_ref, hs_ref, dinv_ref):
    d = deg_ref[...]
    deg_row = d[0:1, :] + d[1:2, :] + 1.0
    dinv = lax.rsqrt(deg_row).reshape(N_PAD, 1)
    dinv_ref[...] = dinv
    hs_ref[...] = h_ref[...] * dinv


def _tc_mid_body(acc_ref, hs_ref, dinv_ref, b1_ref, w2_ref, o_ref):
    dinv = dinv_ref[...]
    a = acc_ref[...]
    out1 = (a[:, :32] + a[:, 32:] + hs_ref[...]) * dinv + b1_ref[...]
    out1 = jnp.maximum(out1, 0.0)
    h2 = jnp.dot(out1, w2_ref[...], preferred_element_type=jnp.float32)
    row = lax.broadcasted_iota(jnp.int32, (N_PAD, 1), 0)
    o_ref[...] = jnp.where(row < N_NODES, h2 * dinv, 0.0)


def _tc_out_body(acc_ref, hs_ref, dinv_ref, b2_ref, o_ref):
    a = acc_ref[...]
    full = (a[:, :16] + a[:, 16:] + hs_ref[...]) * dinv_ref[...]
    logits = full[:N_NODES, :7] + b2_ref[...]
    m = jnp.max(logits, axis=1, keepdims=True)
    e = jnp.exp(logits - m)
    o_ref[...] = e / jnp.sum(e, axis=1, keepdims=True)


def kernel(x, edge_index, W1, b1, W2, b2):
    x = x.astype(jnp.float32)
    src = edge_index[0].astype(jnp.int32)
    dst = edge_index[1].astype(jnp.int32)
    n_edges = src.shape[0]

    per_w = CHUNK * ((n_edges + N_WORKERS * CHUNK - 1) // (N_WORKERS * CHUNK))
    if (per_w // CHUNK) % _NBUF:
        per_w += CHUNK * (_NBUF - (per_w // CHUNK) % _NBUF)
    e_pad = per_w * N_WORKERS
    n_chunks = e_pad // CHUNK
    pad = PAD_IDX + (jnp.arange(e_pad - n_edges, dtype=jnp.int32)
                     % (N_PAD - N_NODES))
    src_c = jnp.concatenate([src, pad]).reshape(n_chunks, CHUNK)
    dst_c = jnp.concatenate([dst, pad]).reshape(n_chunks, CHUNK)

    w2_p = jnp.pad(W2.astype(jnp.float32), ((0, 0), (0, 16 - W2.shape[1])))

    agg16 = _make_sc_agg(n_chunks, 16)
    agg32 = _make_sc_agg(n_chunks, 32)

    deg_parts = _make_sc_deg(n_chunks)(dst_c)
    h1 = pl.pallas_call(
        _tc_matmul_body,
        out_shape=jax.ShapeDtypeStruct((N_PAD, 32), jnp.float32),
    )(x, W1.astype(jnp.float32))

    hs1, dinv = pl.pallas_call(
        _tc_scale_body,
        out_shape=(jax.ShapeDtypeStruct((N_PAD, 32), jnp.float32),
                   jax.ShapeDtypeStruct((N_PAD, 1), jnp.float32)),
    )(deg_parts, h1)

    acc1 = agg32(hs1, src_c, dst_c)

    hs2 = pl.pallas_call(
        _tc_mid_body,
        out_shape=jax.ShapeDtypeStruct((N_PAD, 16), jnp.float32),
    )(acc1, hs1, dinv, b1.astype(jnp.float32).reshape(1, 32), w2_p)

    acc2 = agg16(hs2, src_c, dst_c)

    out = pl.pallas_call(
        _tc_out_body,
        out_shape=jax.ShapeDtypeStruct((N_NODES, 7), jnp.float32),
    )(acc2, hs2, dinv, b2.astype(jnp.float32).reshape(1, 7))
    return out


# speedup vs baseline: 61.9375x; 1.0383x over previous
import functools

import jax
import jax.numpy as jnp
from jax import lax
from jax.experimental import pallas as pl
from jax.experimental.pallas import tpu as pltpu
from jax.experimental.pallas import tpu_sc as plsc

N_NODES = 10000
N_PAD = 10112
PAD_IDX = N_NODES
CHUNK = 128
N_CORES = 2
N_SUB = 16
N_WORKERS = N_CORES * N_SUB
ROWS_PER_SUB = N_PAD // N_SUB


_NBUF = 8


def _zero_acc_slice(zbuf, acc, sid, d):
    @pl.loop(0, CHUNK)
    def _(i):
        if d == 16:
            zbuf[i, :] = jnp.zeros((16,), jnp.float32)
        else:
            for c in range(d // 16):
                zbuf[i, pl.ds(c * 16, 16)] = jnp.zeros((16,), jnp.float32)

    r0 = sid * ROWS_PER_SUB
    nfull = ROWS_PER_SUB // CHUNK
    for k in range(nfull):
        pltpu.sync_copy(zbuf, acc.at[pl.ds(r0 + k * CHUNK, CHUNK)])
    rem = ROWS_PER_SUB - nfull * CHUNK
    if rem:
        pltpu.sync_copy(zbuf.at[pl.ds(0, rem)],
                        acc.at[pl.ds(r0 + nfull * CHUNK, rem)])


def _sc_agg_body(ch_per_w, d, hs_hbm, src_hbm, dst_hbm, out_hbm,
                 sidx, didx, rows, gsems, ssems, zbuf, acc):
    cid = lax.axis_index("c")
    sid = lax.axis_index("s")
    wid = cid * N_SUB + sid

    base = wid * ch_per_w
    pltpu.async_copy(src_hbm.at[pl.ds(base, ch_per_w)], sidx, gsems[0])
    pltpu.async_copy(dst_hbm.at[pl.ds(base, ch_per_w)], didx, gsems[1])
    _zero_acc_slice(zbuf, acc, sid, d)
    pltpu.make_async_copy(src_hbm.at[pl.ds(base, ch_per_w)], sidx,
                          gsems[0]).wait()
    pltpu.make_async_copy(dst_hbm.at[pl.ds(base, ch_per_w)], didx,
                          gsems[1]).wait()
    plsc.subcore_barrier()

    def gather(j, b):
        pltpu.async_copy(hs_hbm.at[sidx.at[j]], rows[b], gsems[b])

    def wait_gather(b):
        pltpu.make_async_copy(hs_hbm.at[sidx.at[0]], rows[b], gsems[b]).wait()

    def scatter(j, b):
        pltpu.async_copy(rows[b], acc.at[didx.at[j]], ssems[b], add=True)

    def wait_scatter(b):
        pltpu.make_async_copy(rows[b], acc.at[didx.at[0]], ssems[b]).wait()

    for b in range(_NBUF):
        gather(b, b)

    @pl.loop(0, ch_per_w - _NBUF, step=_NBUF)
    def _(j):
        for b in range(_NBUF):
            wait_gather(b)
            scatter(j + b, b)
        for b in range(_NBUF):
            wait_scatter(b)
            gather(j + _NBUF + b, b)

    for b in range(_NBUF):
        wait_gather(b)
        scatter(ch_per_w - _NBUF + b, b)
    for b in range(_NBUF):
        wait_scatter(b)

    plsc.subcore_barrier()
    pltpu.sync_copy(acc.at[pl.ds(sid * ROWS_PER_SUB, ROWS_PER_SUB)],
                    out_hbm.at[pl.ds(sid * ROWS_PER_SUB, ROWS_PER_SUB),
                               pl.ds(cid * d, d)])


def _make_sc_agg(n_chunks, d):
    ch_per_w = n_chunks // N_WORKERS
    mesh = plsc.VectorSubcoreMesh(core_axis_name="c", subcore_axis_name="s")
    return pl.kernel(
        functools.partial(_sc_agg_body, ch_per_w, d),
        name=f"sc_agg{d}",
        out_type=jax.ShapeDtypeStruct((N_PAD, 2 * d), jnp.float32),
        mesh=mesh,
        scratch_types=[
            pltpu.VMEM((ch_per_w, CHUNK), jnp.int32),
            pltpu.VMEM((ch_per_w, CHUNK), jnp.int32),
            [pltpu.VMEM((CHUNK, d), jnp.float32) for _ in range(_NBUF)],
            [pltpu.SemaphoreType.DMA for _ in range(_NBUF)],
            [pltpu.SemaphoreType.DMA for _ in range(_NBUF)],
            pltpu.VMEM((CHUNK, d), jnp.float32),
            pltpu.VMEM_SHARED((N_PAD, d), jnp.float32),
        ],
        compiler_params=pltpu.CompilerParams(use_tc_tiling_on_sc=False,
                                             needs_layout_passes=False),
    )


def _sc_deg_body(ch_per_w, dst_hbm, out_hbm, didx, ones_v, zbuf, cbuf,
                 cvec, acc, sem):
    cid = lax.axis_index("c")
    sid = lax.axis_index("s")
    wid = cid * N_SUB + sid

    pltpu.async_copy(dst_hbm.at[pl.ds(wid * ch_per_w, ch_per_w)], didx, sem)

    @pl.loop(0, CHUNK)
    def _(i):
        ones_v[i, :] = jnp.full((16,), 1.0, jnp.float32)

    _zero_acc_slice(zbuf, acc, sid, 16)
    pltpu.make_async_copy(dst_hbm.at[pl.ds(wid * ch_per_w, ch_per_w)], didx,
                          sem).wait()
    plsc.subcore_barrier()

    @pl.loop(0, ch_per_w)
    def _(j):
        pltpu.async_copy(ones_v, acc.at[didx.at[j]], sem, add=True)

    @pl.loop(0, ch_per_w)
    def _(j):
        pltpu.make_async_copy(ones_v, acc.at[didx.at[0]], sem).wait()

    plsc.subcore_barrier()
    pltpu.sync_copy(acc.at[pl.ds(sid * ROWS_PER_SUB, ROWS_PER_SUB)], cbuf)
    zero16 = jnp.zeros((16,), jnp.int32)
    base = lax.iota(jnp.int32, 16)

    @pl.loop(0, ROWS_PER_SUB, step=16)
    def _(i):
        v = plsc.load_gather(cbuf, [base + i, zero16])
        cvec.at[pl.ds(i, 16)][...] = v

    pltpu.sync_copy(cvec, out_hbm.at[cid, pl.ds(sid * ROWS_PER_SUB,
                                                ROWS_PER_SUB)])


def _make_sc_deg(n_chunks):
    ch_per_w = n_chunks // N_WORKERS
    mesh = plsc.VectorSubcoreMesh(core_axis_name="c", subcore_axis_name="s")
    return pl.kernel(
        functools.partial(_sc_deg_body, ch_per_w),
        out_type=jax.ShapeDtypeStruct((N_CORES, N_PAD), jnp.float32),
        mesh=mesh,
        scratch_types=[
            pltpu.VMEM((ch_per_w, CHUNK), jnp.int32),
            pltpu.VMEM((CHUNK, 16), jnp.float32),
            pltpu.VMEM((CHUNK, 16), jnp.float32),
            pltpu.VMEM((ROWS_PER_SUB, 16), jnp.float32),
            pltpu.VMEM((ROWS_PER_SUB,), jnp.float32),
            pltpu.VMEM_SHARED((N_PAD, 16), jnp.float32),
            pltpu.SemaphoreType.DMA,
        ],
        compiler_params=pltpu.CompilerParams(use_tc_tiling_on_sc=False,
                                             needs_layout_passes=False),
    )


def _tc_matmul_body(x_ref, w_ref, o_ref):
    h = jnp.dot(x_ref[...], w_ref[...], preferred_element_type=jnp.float32)
    o_ref[...] = jnp.concatenate(
        [h, jnp.zeros((N_PAD - N_NODES, h.shape[1]), jnp.float32)], axis=0)


def _tc_scale_body(deg_ref, h_ref, hs_ref, dinv_ref):
    d = deg_ref[...]
    deg_row = d[0:1, :] + d[1:2, :] + 1.0
    dinv = lax.rsqrt(deg_row).reshape(N_PAD, 1)
    dinv_ref[...] = dinv
    hs_ref[...] = h_ref[...] * dinv


def _tc_mid_body(acc_ref, hs_ref, dinv_ref, b1_ref, w2_ref, o_ref):
    dinv = dinv_ref[...]
    a = acc_ref[...]
    out1 = (a[:, :32] + a[:, 32:] + hs_ref[...]) * dinv + b1_ref[...]
    out1 = jnp.maximum(out1, 0.0)
    h2 = jnp.dot(out1, w2_ref[...], preferred_element_type=jnp.float32)
    row = lax.broadcasted_iota(jnp.int32, (N_PAD, 1), 0)
    o_ref[...] = jnp.where(row < N_NODES, h2 * dinv, 0.0)


def _tc_out_body(acc_ref, hs_ref, dinv_ref, b2_ref, o_ref):
    a = acc_ref[...]
    full = (a[:, :16] + a[:, 16:] + hs_ref[...]) * dinv_ref[...]
    logits = full[:N_NODES, :7] + b2_ref[...]
    m = jnp.max(logits, axis=1, keepdims=True)
    e = jnp.exp(logits - m)
    o_ref[...] = e / jnp.sum(e, axis=1, keepdims=True)


def kernel(x, edge_index, W1, b1, W2, b2):
    x = x.astype(jnp.float32)
    src = edge_index[0].astype(jnp.int32)
    dst = edge_index[1].astype(jnp.int32)
    n_edges = src.shape[0]

    per_w = CHUNK * ((n_edges + N_WORKERS * CHUNK - 1) // (N_WORKERS * CHUNK))
    if (per_w // CHUNK) % _NBUF:
        per_w += CHUNK * (_NBUF - (per_w // CHUNK) % _NBUF)
    e_pad = per_w * N_WORKERS
    n_chunks = e_pad // CHUNK
    pad = PAD_IDX + (jnp.arange(e_pad - n_edges, dtype=jnp.int32)
                     % (N_PAD - N_NODES))
    src_c = jnp.concatenate([src, pad]).reshape(n_chunks, CHUNK)
    dst_c = jnp.concatenate([dst, pad]).reshape(n_chunks, CHUNK)

    w2_p = jnp.pad(W2.astype(jnp.float32), ((0, 0), (0, 16 - W2.shape[1])))

    agg16 = _make_sc_agg(n_chunks, 16)
    agg32 = _make_sc_agg(n_chunks, 32)

    deg_parts = _make_sc_deg(n_chunks)(dst_c)
    h1 = pl.pallas_call(
        _tc_matmul_body,
        out_shape=jax.ShapeDtypeStruct((N_PAD, 32), jnp.float32),
    )(x, W1.astype(jnp.float32))

    hs1, dinv = pl.pallas_call(
        _tc_scale_body,
        out_shape=(jax.ShapeDtypeStruct((N_PAD, 32), jnp.float32),
                   jax.ShapeDtypeStruct((N_PAD, 1), jnp.float32)),
    )(deg_parts, h1)

    acc1 = agg32(hs1, src_c, dst_c)

    hs2 = pl.pallas_call(
        _tc_mid_body,
        out_shape=jax.ShapeDtypeStruct((N_PAD, 16), jnp.float32),
    )(acc1, hs1, dinv, b1.astype(jnp.float32).reshape(1, 32), w2_p)

    acc2 = agg16(hs2, src_c, dst_c)

    out = pl.pallas_call(
        _tc_out_body,
        out_shape=jax.ShapeDtypeStruct((N_NODES, 7), jnp.float32),
    )(acc2, hs2, dinv, b2.astype(jnp.float32).reshape(1, 7))
    return out


# speedup vs baseline: 61.9935x; 1.0009x over previous
import functools

import jax
import jax.numpy as jnp
from jax import lax
from jax.experimental import pallas as pl
from jax.experimental.pallas import tpu as pltpu
from jax.experimental.pallas import tpu_sc as plsc

N_NODES = 10000
N_PAD = 10112
PAD_IDX = N_NODES
CHUNK = 128
N_CORES = 2
N_SUB = 16
N_WORKERS = N_CORES * N_SUB
ROWS_PER_SUB = N_PAD // N_SUB


_NBUF = 8


def _zero_acc_slice(zbuf, acc, sid, d):
    @pl.loop(0, CHUNK)
    def _(i):
        if d == 16:
            zbuf[i, :] = jnp.zeros((16,), jnp.float32)
        else:
            for c in range(d // 16):
                zbuf[i, pl.ds(c * 16, 16)] = jnp.zeros((16,), jnp.float32)

    r0 = sid * ROWS_PER_SUB
    nfull = ROWS_PER_SUB // CHUNK
    for k in range(nfull):
        pltpu.sync_copy(zbuf, acc.at[pl.ds(r0 + k * CHUNK, CHUNK)])
    rem = ROWS_PER_SUB - nfull * CHUNK
    if rem:
        pltpu.sync_copy(zbuf.at[pl.ds(0, rem)],
                        acc.at[pl.ds(r0 + nfull * CHUNK, rem)])


def _sc_agg_body(ch_per_w, d, hs_hbm, src_hbm, dst_hbm, out_hbm,
                 sidx, didx, rows, gsems, ssems, zbuf, acc):
    cid = lax.axis_index("c")
    sid = lax.axis_index("s")
    wid = cid * N_SUB + sid

    base = wid * ch_per_w
    pltpu.async_copy(src_hbm.at[pl.ds(base, ch_per_w)], sidx, gsems[0])
    pltpu.async_copy(dst_hbm.at[pl.ds(base, ch_per_w)], didx, gsems[1])
    _zero_acc_slice(zbuf, acc, sid, d)
    pltpu.make_async_copy(src_hbm.at[pl.ds(base, ch_per_w)], sidx,
                          gsems[0]).wait()
    pltpu.make_async_copy(dst_hbm.at[pl.ds(base, ch_per_w)], didx,
                          gsems[1]).wait()
    plsc.subcore_barrier()

    def gather(j, b):
        pltpu.async_copy(hs_hbm.at[sidx.at[j]], rows[b], gsems[b])

    def wait_gather(b):
        pltpu.make_async_copy(hs_hbm.at[sidx.at[0]], rows[b], gsems[b]).wait()

    def scatter(j, b):
        pltpu.async_copy(rows[b], acc.at[didx.at[j]], ssems[b], add=True)

    def wait_scatter(b):
        pltpu.make_async_copy(rows[b], acc.at[didx.at[0]], ssems[b]).wait()

    for b in range(_NBUF):
        gather(b, b)

    @pl.loop(0, ch_per_w - _NBUF, step=_NBUF)
    def _(j):
        for b in range(_NBUF):
            wait_gather(b)
            scatter(j + b, b)
        for b in range(_NBUF):
            wait_scatter(b)
            gather(j + _NBUF + b, b)

    for b in range(_NBUF):
        wait_gather(b)
        scatter(ch_per_w - _NBUF + b, b)
    for b in range(_NBUF):
        wait_scatter(b)

    plsc.subcore_barrier()
    pltpu.sync_copy(acc.at[pl.ds(sid * ROWS_PER_SUB, ROWS_PER_SUB)],
                    out_hbm.at[pl.ds(sid * ROWS_PER_SUB, ROWS_PER_SUB),
                               pl.ds(cid * d, d)])


def _make_sc_agg(n_chunks, d):
    ch_per_w = n_chunks // N_WORKERS
    mesh = plsc.VectorSubcoreMesh(core_axis_name="c", subcore_axis_name="s")
    return pl.kernel(
        functools.partial(_sc_agg_body, ch_per_w, d),
        name=f"sc_agg{d}",
        out_type=jax.ShapeDtypeStruct((N_PAD, 2 * d), jnp.float32),
        mesh=mesh,
        scratch_types=[
            pltpu.VMEM((ch_per_w, CHUNK), jnp.int32),
            pltpu.VMEM((ch_per_w, CHUNK), jnp.int32),
            [pltpu.VMEM((CHUNK, d), jnp.float32) for _ in range(_NBUF)],
            [pltpu.SemaphoreType.DMA for _ in range(_NBUF)],
            [pltpu.SemaphoreType.DMA for _ in range(_NBUF)],
            pltpu.VMEM((CHUNK, d), jnp.float32),
            pltpu.VMEM_SHARED((N_PAD, d), jnp.float32),
        ],
        compiler_params=pltpu.CompilerParams(use_tc_tiling_on_sc=False,
                                             needs_layout_passes=False),
    )


def _sc_deg_body(ch_per_w, dst_hbm, out_hbm, didx, ones_v, zbuf, cbuf,
                 cvec, acc, sem):
    cid = lax.axis_index("c")
    sid = lax.axis_index("s")
    wid = cid * N_SUB + sid

    pltpu.async_copy(dst_hbm.at[pl.ds(wid * ch_per_w, ch_per_w)], didx, sem)

    @pl.loop(0, CHUNK)
    def _(i):
        ones_v[i, :] = jnp.full((16,), 1.0, jnp.float32)

    _zero_acc_slice(zbuf, acc, sid, 16)
    pltpu.make_async_copy(dst_hbm.at[pl.ds(wid * ch_per_w, ch_per_w)], didx,
                          sem).wait()
    plsc.subcore_barrier()

    @pl.loop(0, ch_per_w)
    def _(j):
        pltpu.async_copy(ones_v, acc.at[didx.at[j]], sem, add=True)

    @pl.loop(0, ch_per_w)
    def _(j):
        pltpu.make_async_copy(ones_v, acc.at[didx.at[0]], sem).wait()

    plsc.subcore_barrier()
    pltpu.sync_copy(acc.at[pl.ds(sid * ROWS_PER_SUB, ROWS_PER_SUB)], cbuf)
    zero16 = jnp.zeros((16,), jnp.int32)
    base = lax.iota(jnp.int32, 16)

    @pl.loop(0, ROWS_PER_SUB, step=16)
    def _(i):
        v = plsc.load_gather(cbuf, [base + i, zero16])
        cvec.at[pl.ds(i, 16)][...] = v

    pltpu.sync_copy(cvec, out_hbm.at[cid, pl.ds(sid * ROWS_PER_SUB,
                                                ROWS_PER_SUB)])


def _make_sc_deg(n_chunks):
    ch_per_w = n_chunks // N_WORKERS
    mesh = plsc.VectorSubcoreMesh(core_axis_name="c", subcore_axis_name="s")
    return pl.kernel(
        functools.partial(_sc_deg_body, ch_per_w),
        out_type=jax.ShapeDtypeStruct((N_CORES, N_PAD), jnp.float32),
        mesh=mesh,
        scratch_types=[
            pltpu.VMEM((ch_per_w, CHUNK), jnp.int32),
            pltpu.VMEM((CHUNK, 16), jnp.float32),
            pltpu.VMEM((CHUNK, 16), jnp.float32),
            pltpu.VMEM((ROWS_PER_SUB, 16), jnp.float32),
            pltpu.VMEM((ROWS_PER_SUB,), jnp.float32),
            pltpu.VMEM_SHARED((N_PAD, 16), jnp.float32),
            pltpu.SemaphoreType.DMA,
        ],
        compiler_params=pltpu.CompilerParams(use_tc_tiling_on_sc=False,
                                             needs_layout_passes=False),
    )


def _tc_matmul_body(x_ref, w_ref, o_ref):
    h = jnp.dot(x_ref[...], w_ref[...], preferred_element_type=jnp.float32)
    o_ref[...] = jnp.concatenate(
        [h, jnp.zeros((N_PAD - N_NODES, h.shape[1]), jnp.float32)], axis=0)


def _tc_scale_body(deg_ref, h_ref, hs_ref, dinv_ref):
    d = deg_ref[...]
    deg_row = d[0:1, :] + d[1:2, :] + 1.0
    dinv = lax.rsqrt(deg_row).reshape(N_PAD, 1)
    dinv_ref[...] = dinv
    hs_ref[...] = h_ref[...] * dinv


def _tc_mid_body(acc_ref, hs_ref, dinv_ref, b1_ref, w2_ref, o_ref):
    dinv = dinv_ref[...]
    a = acc_ref[...]
    out1 = (a[:, :32] + a[:, 32:] + hs_ref[...]) * dinv + b1_ref[...]
    out1 = jnp.maximum(out1, 0.0)
    h2 = jnp.dot(out1, w2_ref[...], preferred_element_type=jnp.float32)
    row = lax.broadcasted_iota(jnp.int32, (N_PAD, 1), 0)
    o_ref[...] = jnp.where(row < N_NODES, h2 * dinv, 0.0)


def _tc_out_body(acc_ref, hs_ref, dinv_ref, b2_ref, o_ref):
    a = acc_ref[...]
    full = (a[:, :16] + a[:, 16:] + hs_ref[...]) * dinv_ref[...]
    logits = full[:N_NODES, :7] + b2_ref[...]
    m = jnp.max(logits, axis=1, keepdims=True)
    e = jnp.exp(logits - m)
    o_ref[...] = e / jnp.sum(e, axis=1, keepdims=True)


def kernel(x, edge_index, W1, b1, W2, b2):
    x = x.astype(jnp.float32)
    src = edge_index[0].astype(jnp.int32)
    dst = edge_index[1].astype(jnp.int32)
    n_edges = src.shape[0]

    per_w = CHUNK * ((n_edges + N_WORKERS * CHUNK - 1) // (N_WORKERS * CHUNK))
    if (per_w // CHUNK) % _NBUF:
        per_w += CHUNK * (_NBUF - (per_w // CHUNK) % _NBUF)
    e_pad = per_w * N_WORKERS
    n_chunks = e_pad // CHUNK
    pad = PAD_IDX + (jnp.arange(e_pad - n_edges, dtype=jnp.int32)
                     % (N_PAD - N_NODES))
    dst_c = jnp.concatenate([dst, pad]).reshape(n_chunks, CHUNK)

    w2_p = jnp.pad(W2.astype(jnp.float32), ((0, 0), (0, 16 - W2.shape[1])))

    agg16 = _make_sc_agg(n_chunks, 16)
    agg32 = _make_sc_agg(n_chunks, 32)

    deg_parts = _make_sc_deg(n_chunks)(dst_c)
    src_c = jnp.concatenate([src, pad]).reshape(n_chunks, CHUNK)
    h1 = pl.pallas_call(
        _tc_matmul_body,
        out_shape=jax.ShapeDtypeStruct((N_PAD, 32), jnp.float32),
    )(x, W1.astype(jnp.float32))

    hs1, dinv = pl.pallas_call(
        _tc_scale_body,
        out_shape=(jax.ShapeDtypeStruct((N_PAD, 32), jnp.float32),
                   jax.ShapeDtypeStruct((N_PAD, 1), jnp.float32)),
    )(deg_parts, h1)

    acc1 = agg32(hs1, src_c, dst_c)

    hs2 = pl.pallas_call(
        _tc_mid_body,
        out_shape=jax.ShapeDtypeStruct((N_PAD, 16), jnp.float32),
    )(acc1, hs1, dinv, b1.astype(jnp.float32).reshape(1, 32), w2_p)

    acc2 = agg16(hs2, src_c, dst_c)

    out = pl.pallas_call(
        _tc_out_body,
        out_shape=jax.ShapeDtypeStruct((N_NODES, 7), jnp.float32),
    )(acc2, hs2, dinv, b2.astype(jnp.float32).reshape(1, 7))
    return out
